# baseline (speedup 1.0000x reference)
import functools

import jax
import jax.numpy as jnp
from jax import lax
from jax.experimental import pallas as pl
from jax.experimental.pallas import tpu as pltpu
from jax.experimental.pallas import tpu_sc as plsc

S, D, F, E, K = 2048, 768, 1024, 8, 2
T = 256
NB = (S * K) // T + (E - 1)
NPAD = NB * T
MB = 32
C = 256
NCHUNK = S // C
NW = 32
TOK_W = S // NW
LANES = 16
D2 = D // 2


def _router_body(x_ref, wg_ref, slot0_ref, slot1_ref, w0_ref, w1_ref,
                 meta_ref, xbf_ref):
    x = x_ref[...]
    wg = wg_ref[...]
    logits = lax.dot_general(x, wg, (((1,), (1,)), ((), ())),
                             preferred_element_type=jnp.float32)
    eio = lax.broadcasted_iota(jnp.int32, (S, E), 1)
    m0 = jnp.max(logits, axis=1, keepdims=True)
    i0 = jnp.min(jnp.where(logits == m0, eio, E), axis=1, keepdims=True)
    l2 = jnp.where(eio == i0, -jnp.inf, logits)
    m1 = jnp.max(l2, axis=1, keepdims=True)
    i1 = jnp.min(jnp.where(l2 == m1, eio, E), axis=1, keepdims=True)
    w0 = 1.0 / (1.0 + jnp.exp(m1 - m0))
    w1 = 1.0 - w0
    w0_ref[...] = jnp.broadcast_to(w0, (S, LANES))
    w1_ref[...] = jnp.broadcast_to(w1, (S, LANES))
    bits = lax.bitcast_convert_type(x, jnp.uint32)
    blo, bhi = bits[:, :D2], bits[:, D2:]
    rlo = (blo + 0x7FFF + ((blo >> 16) & 1)) >> 16
    rhi = ((bhi + 0x7FFF + ((bhi >> 16) & 1)) >> 16) << 16
    xbf_ref[...] = lax.bitcast_convert_type(rhi | rlo, jnp.int32)

    oh0 = (eio == i0).astype(jnp.float32)
    oh1 = (eio == i1).astype(jnp.float32)
    cnt0 = jnp.sum(oh0, axis=0, keepdims=True)
    cnt = cnt0 + jnp.sum(oh1, axis=0, keepdims=True)
    nblk = jnp.floor((cnt + (T - 1)) * (1.0 / T))
    upper = (lax.broadcasted_iota(jnp.int32, (E, E), 0)
             <= lax.broadcasted_iota(jnp.int32, (E, E), 1)).astype(jnp.float32)
    inc = lax.dot_general(nblk, upper, (((1,), (0,)), ((), ())),
                          preferred_element_type=jnp.float32)
    gs = (inc - nblk) * T

    bio = lax.broadcasted_iota(jnp.int32, (MB, E), 0)
    inc_i = inc.astype(jnp.int32)
    be = jnp.sum((bio >= inc_i).astype(jnp.int32),
                 axis=1, keepdims=True)
    be_p = jnp.sum(((bio - 1) >= inc_i).astype(jnp.int32),
                   axis=1, keepdims=True)
    be = jnp.minimum(be, E - 1)
    be_p = jnp.minimum(be_p, E - 1)
    used = inc[:, E - 1:E]
    bcol = lax.broadcasted_iota(jnp.int32, (MB, 1), 0)
    act = (bcol.astype(jnp.float32) < used).astype(jnp.int32)
    sw = jnp.where((bcol > 0) & (be != be_p), 1, 0) * act
    mtri = (lax.broadcasted_iota(jnp.int32, (MB, MB), 0)
            >= lax.broadcasted_iota(jnp.int32, (MB, MB), 1)).astype(
                jnp.float32)
    run_id = lax.dot_general(mtri, sw.astype(jnp.float32),
                             (((1,), (0,)), ((), ())),
                             preferred_element_type=jnp.float32)
    par = (run_id - 2.0 * jnp.floor(run_id * 0.5)).astype(jnp.int32)
    first = jnp.where((bcol == 0) | (sw == 1), 1, 0) * act
    has = jnp.broadcast_to((nblk > 0.5), (MB, E))
    nxte = jnp.min(jnp.where((eio[:MB] > be) & has, eio[:MB], E),
                   axis=1, keepdims=True)
    issue = first * jnp.where(nxte < E, 1, 0)
    nxte = jnp.minimum(nxte, E - 1)
    meta = jnp.concatenate(
        [act, par, first, issue, nxte, be, be, be], axis=1)
    meta_ref[...] = meta

    ltri = (lax.broadcasted_iota(jnp.int32, (C, C), 0)
            > lax.broadcasted_iota(jnp.int32, (C, C), 1)).astype(jnp.float32)
    carry0 = jnp.zeros((1, E), jnp.float32)
    carry1 = cnt0
    for c in range(NCHUNK):
        sl = slice(c * C, (c + 1) * C)
        o0 = oh0[sl]
        o1 = oh1[sl]
        p0 = carry0 + lax.dot_general(ltri, o0, (((1,), (0,)), ((), ())),
                                      preferred_element_type=jnp.float32)
        p1 = carry1 + lax.dot_general(ltri, o1, (((1,), (0,)), ((), ())),
                                      preferred_element_type=jnp.float32)
        carry0 = carry0 + jnp.sum(o0, axis=0, keepdims=True)
        carry1 = carry1 + jnp.sum(o1, axis=0, keepdims=True)
        slot0_ref[sl, :] = jnp.sum((p0 + gs) * o0, axis=1,
                                   keepdims=True).astype(jnp.int32)
        slot1_ref[sl, :] = jnp.sum((p1 + gs) * o1, axis=1,
                                   keepdims=True).astype(jnp.int32)


_router_call = pl.pallas_call(
    _router_body,
    out_shape=(
        jax.ShapeDtypeStruct((S, 1), jnp.int32),
        jax.ShapeDtypeStruct((S, 1), jnp.int32),
        jax.ShapeDtypeStruct((S, LANES), jnp.float32),
        jax.ShapeDtypeStruct((S, LANES), jnp.float32),
        jax.ShapeDtypeStruct((MB, 8), jnp.int32),
        jax.ShapeDtypeStruct((S, D2), jnp.int32),
    ),
)


@functools.lru_cache(maxsize=None)
def _get_sc_mesh():
    return plsc.VectorSubcoreMesh(core_axis_name="c", subcore_axis_name="s")


def _dispatch_body(x_hbm, slot0_hbm, slot1_hbm, xs_hbm, rows_v, idx0_v,
                   idx1_v, sem):
    wid = lax.axis_index("s") * 2 + lax.axis_index("c")
    base = wid * TOK_W
    pltpu.sync_copy(x_hbm.at[pl.ds(base, TOK_W)], rows_v)
    pltpu.sync_copy(slot0_hbm.at[pl.ds(base, TOK_W)], idx0_v)
    pltpu.sync_copy(slot1_hbm.at[pl.ds(base, TOK_W)], idx1_v)
    pltpu.async_copy(rows_v, xs_hbm.at[idx0_v], sem).wait()
    pltpu.async_copy(rows_v, xs_hbm.at[idx1_v], sem).wait()


@functools.lru_cache(maxsize=None)
def _get_dispatch():
    return pl.kernel(
        _dispatch_body,
        out_type=jax.ShapeDtypeStruct((NPAD, D2), jnp.int32),
        mesh=_get_sc_mesh(),
        scratch_types=[
            pltpu.VMEM((TOK_W, D2), jnp.int32),
            pltpu.VMEM((TOK_W,), jnp.int32),
            pltpu.VMEM((TOK_W,), jnp.int32),
            pltpu.SemaphoreType.DMA,
        ],
    )


F2 = F // 2
D_2 = D // 2


def _wcopies(w1_hbm, w3_hbm, b3_hbm, w2_hbm, w1b, w3b, b3b, w2b, sems,
             e, slot):
    fa, fb = pl.ds(0, F2), pl.ds(F2, F2)
    da, db = pl.ds(0, D_2), pl.ds(D_2, D_2)
    return (
        pltpu.make_async_copy(w1_hbm.at[e, fa], w1b.at[slot, fa],
                              sems.at[slot, 0]),
        pltpu.make_async_copy(w1_hbm.at[e, fb], w1b.at[slot, fb],
                              sems.at[slot, 1]),
        pltpu.make_async_copy(w3_hbm.at[e, fa], w3b.at[slot, fa],
                              sems.at[slot, 2]),
        pltpu.make_async_copy(w3_hbm.at[e, fb], w3b.at[slot, fb],
                              sems.at[slot, 3]),
        pltpu.make_async_copy(b3_hbm.at[e], b3b.at[slot], sems.at[slot, 4]),
        pltpu.make_async_copy(w2_hbm.at[e, da], w2b.at[slot, da],
                              sems.at[slot, 5]),
        pltpu.make_async_copy(w2_hbm.at[e, db], w2b.at[slot, db],
                              sems.at[slot, 6]),
    )


def _ffn_body(meta_ref, xs_ref, w1_hbm, w3_hbm, b3_hbm, w2_hbm, eo_ref,
              w1b, w3b, b3b, w2b, sems):
    b = pl.program_id(0)
    act = meta_ref[b, 0]
    par = meta_ref[b, 1]
    first = meta_ref[b, 2]
    issue = meta_ref[b, 3]
    nxte = meta_ref[b, 4]
    e_cur = meta_ref[b, 5]

    @pl.when(b == 0)
    def _():
        for cp in _wcopies(w1_hbm, w3_hbm, b3_hbm, w2_hbm,
                           w1b, w3b, b3b, w2b, sems, e_cur, 0):
            cp.start()

    @pl.when(issue == 1)
    def _():
        for cp in _wcopies(w1_hbm, w3_hbm, b3_hbm, w2_hbm,
                           w1b, w3b, b3b, w2b, sems, nxte, 1 - par):
            cp.start()

    @pl.when(first == 1)
    def _():
        for cp in _wcopies(w1_hbm, w3_hbm, b3_hbm, w2_hbm,
                           w1b, w3b, b3b, w2b, sems, e_cur, par):
            cp.wait()

    @pl.when(act == 1)
    def _():
        v = xs_ref[...]
        xlo = lax.bitcast_convert_type(v << 16, jnp.float32).astype(
            jnp.bfloat16)
        xhi = lax.bitcast_convert_type(
            v & jnp.int32(-65536), jnp.float32).astype(jnp.bfloat16)
        w1c = w1b[par].astype(jnp.bfloat16)
        w3c = w3b[par].astype(jnp.bfloat16)
        w2c = w2b[par].astype(jnp.bfloat16)
        cdim = (((1,), (1,)), ((), ()))
        h1 = (lax.dot_general(xlo, w1c[:, :D2], cdim,
                              preferred_element_type=jnp.float32)
              + lax.dot_general(xhi, w1c[:, D2:], cdim,
                                preferred_element_type=jnp.float32))
        h3 = (lax.dot_general(xlo, w3c[:, :D2], cdim,
                              preferred_element_type=jnp.float32)
              + lax.dot_general(xhi, w3c[:, D2:], cdim,
                                preferred_element_type=jnp.float32))
        h3 = h3 + b3b[par]
        h = (h1 * lax.logistic(h1) * h3).astype(jnp.bfloat16)
        eo_ref[...] = lax.dot_general(h, w2c, cdim,
                                      preferred_element_type=jnp.float32)


_ffn_call = pl.pallas_call(
    _ffn_body,
    grid_spec=pltpu.PrefetchScalarGridSpec(
        num_scalar_prefetch=1,
        grid=(NB,),
        in_specs=[
            pl.BlockSpec((T, D2), lambda b, meta: (b, 0)),
            pl.BlockSpec(memory_space=pl.ANY),
            pl.BlockSpec(memory_space=pl.ANY),
            pl.BlockSpec(memory_space=pl.ANY),
            pl.BlockSpec(memory_space=pl.ANY),
        ],
        out_specs=pl.BlockSpec((T, D), lambda b, meta: (b, 0)),
        scratch_shapes=[
            pltpu.VMEM((2, F, D), jnp.float32),
            pltpu.VMEM((2, F, D), jnp.float32),
            pltpu.VMEM((2, 1, F), jnp.float32),
            pltpu.VMEM((2, D, F), jnp.float32),
            pltpu.SemaphoreType.DMA((2, 7)),
        ],
    ),
    out_shape=jax.ShapeDtypeStruct((NPAD, D), jnp.float32),
)


def _combine_body(eo_hbm, slot0_hbm, slot1_hbm, w0_hbm, w1_hbm, out_hbm,
                  idx0_v, idx1_v, w0_v, w1_v, r0_v, r1_v, sem):
    wid = lax.axis_index("s") * 2 + lax.axis_index("c")
    base = wid * TOK_W
    pltpu.sync_copy(slot0_hbm.at[pl.ds(base, TOK_W)], idx0_v)
    pltpu.sync_copy(slot1_hbm.at[pl.ds(base, TOK_W)], idx1_v)
    pltpu.sync_copy(w0_hbm.at[pl.ds(base, TOK_W)], w0_v)
    pltpu.sync_copy(w1_hbm.at[pl.ds(base, TOK_W)], w1_v)
    H = TOK_W // 2
    ha, hb = pl.ds(0, H), pl.ds(H, H)
    cA0 = pltpu.async_copy(eo_hbm.at[idx0_v.at[ha]], r0_v.at[ha], sem.at[0])
    cA1 = pltpu.async_copy(eo_hbm.at[idx1_v.at[ha]], r1_v.at[ha], sem.at[0])
    cB0 = pltpu.async_copy(eo_hbm.at[idx0_v.at[hb]], r0_v.at[hb], sem.at[1])
    cB1 = pltpu.async_copy(eo_hbm.at[idx1_v.at[hb]], r1_v.at[hb], sem.at[1])

    def body(i, carry):
        wv0 = w0_v[i]
        wv1 = w1_v[i]
        for j in range(D // LANES):
            sl = pl.ds(j * LANES, LANES)
            r0_v[i, sl] = wv0 * r0_v[i, sl] + wv1 * r1_v[i, sl]
        return carry

    cA0.wait()
    cA1.wait()
    lax.fori_loop(0, H, body, 0)
    oA = pltpu.async_copy(r0_v.at[ha], out_hbm.at[pl.ds(base, H)], sem.at[0])
    cB0.wait()
    cB1.wait()
    lax.fori_loop(H, TOK_W, body, 0)
    oA.wait()
    pltpu.sync_copy(r0_v.at[hb], out_hbm.at[pl.ds(base + H, H)])


@functools.lru_cache(maxsize=None)
def _get_combine():
    return pl.kernel(
        _combine_body,
        out_type=jax.ShapeDtypeStruct((S, D), jnp.float32),
        mesh=_get_sc_mesh(),
        compiler_params=pltpu.CompilerParams(needs_layout_passes=False),
        scratch_types=[
            pltpu.VMEM((TOK_W,), jnp.int32),
            pltpu.VMEM((TOK_W,), jnp.int32),
            pltpu.VMEM((TOK_W, LANES), jnp.float32),
            pltpu.VMEM((TOK_W, LANES), jnp.float32),
            pltpu.VMEM((TOK_W, D), jnp.float32),
            pltpu.VMEM((TOK_W, D), jnp.float32),
            pltpu.SemaphoreType.DMA((2,)),
        ],
    )


@jax.jit
def kernel(x, Wg, W1, W3, b3, W2):
    x2 = x.reshape(S, D)
    slot0, slot1, w0b, w1b, meta, xbf = _router_call(x2, Wg)
    slot0 = slot0.reshape(S)
    slot1 = slot1.reshape(S)
    xs = _get_dispatch()(xbf, slot0, slot1)
    eo = _ffn_call(meta, xs, W1, W3, b3.reshape(E, 1, F), W2)
    out = _get_combine()(eo, slot0, slot1, w0b, w1b)
    return out.reshape(1, S, D)


# speedup vs baseline: 1.0388x; 1.0388x over previous
import functools

import jax
import jax.numpy as jnp
from jax import lax
from jax.experimental import pallas as pl
from jax.experimental.pallas import tpu as pltpu
from jax.experimental.pallas import tpu_sc as plsc

S, D, F, E, K = 2048, 768, 1024, 8, 2
T = 256
NB = (S * K) // T + (E - 1)
NPAD = NB * T
MB = 32
C = 256
NCHUNK = S // C
NW = 32
TOK_W = S // NW
LANES = 16
D2 = D // 2


def _router_body(x_ref, wg_ref, slot0_ref, slot1_ref, w0_ref, w1_ref,
                 meta_ref, xbf_ref):
    x = x_ref[...]
    wg = wg_ref[...]
    logits = lax.dot_general(x, wg, (((1,), (1,)), ((), ())),
                             preferred_element_type=jnp.float32)
    eio = lax.broadcasted_iota(jnp.int32, (S, E), 1)
    m0 = jnp.max(logits, axis=1, keepdims=True)
    i0 = jnp.min(jnp.where(logits == m0, eio, E), axis=1, keepdims=True)
    l2 = jnp.where(eio == i0, -jnp.inf, logits)
    m1 = jnp.max(l2, axis=1, keepdims=True)
    i1 = jnp.min(jnp.where(l2 == m1, eio, E), axis=1, keepdims=True)
    w0 = 1.0 / (1.0 + jnp.exp(m1 - m0))
    w1 = 1.0 - w0
    w0_ref[...] = jnp.broadcast_to(w0, (S, LANES))
    w1_ref[...] = jnp.broadcast_to(w1, (S, LANES))
    bits = lax.bitcast_convert_type(x, jnp.uint32)
    blo, bhi = bits[:, :D2], bits[:, D2:]
    rlo = (blo + 0x7FFF + ((blo >> 16) & 1)) >> 16
    rhi = ((bhi + 0x7FFF + ((bhi >> 16) & 1)) >> 16) << 16
    xbf_ref[...] = lax.bitcast_convert_type(rhi | rlo, jnp.int32)

    oh0 = (eio == i0).astype(jnp.float32)
    oh1 = (eio == i1).astype(jnp.float32)
    cnt0 = jnp.sum(oh0, axis=0, keepdims=True)
    cnt = cnt0 + jnp.sum(oh1, axis=0, keepdims=True)
    nblk = jnp.floor((cnt + (T - 1)) * (1.0 / T))
    upper = (lax.broadcasted_iota(jnp.int32, (E, E), 0)
             <= lax.broadcasted_iota(jnp.int32, (E, E), 1)).astype(jnp.float32)
    inc = lax.dot_general(nblk, upper, (((1,), (0,)), ((), ())),
                          preferred_element_type=jnp.float32)
    gs = (inc - nblk) * T

    bio = lax.broadcasted_iota(jnp.int32, (MB, E), 0)
    inc_i = inc.astype(jnp.int32)
    be = jnp.sum((bio >= inc_i).astype(jnp.int32),
                 axis=1, keepdims=True)
    be_p = jnp.sum(((bio - 1) >= inc_i).astype(jnp.int32),
                   axis=1, keepdims=True)
    be = jnp.minimum(be, E - 1)
    be_p = jnp.minimum(be_p, E - 1)
    used = inc[:, E - 1:E]
    bcol = lax.broadcasted_iota(jnp.int32, (MB, 1), 0)
    act = (bcol.astype(jnp.float32) < used).astype(jnp.int32)
    sw = jnp.where((bcol > 0) & (be != be_p), 1, 0) * act
    mtri = (lax.broadcasted_iota(jnp.int32, (MB, MB), 0)
            >= lax.broadcasted_iota(jnp.int32, (MB, MB), 1)).astype(
                jnp.float32)
    run_id = lax.dot_general(mtri, sw.astype(jnp.float32),
                             (((1,), (0,)), ((), ())),
                             preferred_element_type=jnp.float32)
    par = (run_id - 2.0 * jnp.floor(run_id * 0.5)).astype(jnp.int32)
    first = jnp.where((bcol == 0) | (sw == 1), 1, 0) * act
    has = jnp.broadcast_to((nblk > 0.5), (MB, E))
    nxte = jnp.min(jnp.where((eio[:MB] > be) & has, eio[:MB], E),
                   axis=1, keepdims=True)
    issue = first * jnp.where(nxte < E, 1, 0)
    nxte = jnp.minimum(nxte, E - 1)
    meta = jnp.concatenate(
        [act, par, first, issue, nxte, be, be, be], axis=1)
    meta_ref[...] = meta

    ltri = (lax.broadcasted_iota(jnp.int32, (C, C), 0)
            > lax.broadcasted_iota(jnp.int32, (C, C), 1)).astype(jnp.float32)
    carry0 = jnp.zeros((1, E), jnp.float32)
    carry1 = cnt0
    cols0, cols1 = [], []
    for c in range(NCHUNK):
        sl = slice(c * C, (c + 1) * C)
        o0 = oh0[sl]
        o1 = oh1[sl]
        p0 = carry0 + lax.dot_general(ltri, o0, (((1,), (0,)), ((), ())),
                                      preferred_element_type=jnp.float32)
        p1 = carry1 + lax.dot_general(ltri, o1, (((1,), (0,)), ((), ())),
                                      preferred_element_type=jnp.float32)
        carry0 = carry0 + jnp.sum(o0, axis=0, keepdims=True)
        carry1 = carry1 + jnp.sum(o1, axis=0, keepdims=True)
        cols0.append(jnp.sum((p0 + gs) * o0, axis=1, keepdims=True))
        cols1.append(jnp.sum((p1 + gs) * o1, axis=1, keepdims=True))
    slot0_ref[...] = jnp.transpose(
        jnp.concatenate(cols0, axis=1), (1, 0)).astype(jnp.int32)
    slot1_ref[...] = jnp.transpose(
        jnp.concatenate(cols1, axis=1), (1, 0)).astype(jnp.int32)


_router_call = pl.pallas_call(
    _router_body,
    out_shape=(
        jax.ShapeDtypeStruct((NCHUNK, C), jnp.int32),
        jax.ShapeDtypeStruct((NCHUNK, C), jnp.int32),
        jax.ShapeDtypeStruct((S, LANES), jnp.float32),
        jax.ShapeDtypeStruct((S, LANES), jnp.float32),
        jax.ShapeDtypeStruct((MB, 8), jnp.int32),
        jax.ShapeDtypeStruct((S, D2), jnp.int32),
    ),
)


@functools.lru_cache(maxsize=None)
def _get_sc_mesh():
    return plsc.VectorSubcoreMesh(core_axis_name="c", subcore_axis_name="s")


def _dispatch_body(x_hbm, slot0_hbm, slot1_hbm, xs_hbm, rows_v, idx0_v,
                   idx1_v, sem):
    wid = lax.axis_index("s") * 2 + lax.axis_index("c")
    base = wid * TOK_W
    ch = wid // (C // TOK_W)
    off = (wid % (C // TOK_W)) * TOK_W
    pltpu.sync_copy(x_hbm.at[pl.ds(base, TOK_W)], rows_v)
    pltpu.sync_copy(slot0_hbm.at[ch, pl.ds(off, TOK_W)], idx0_v)
    pltpu.sync_copy(slot1_hbm.at[ch, pl.ds(off, TOK_W)], idx1_v)
    pltpu.async_copy(rows_v, xs_hbm.at[idx0_v], sem).wait()
    pltpu.async_copy(rows_v, xs_hbm.at[idx1_v], sem).wait()


@functools.lru_cache(maxsize=None)
def _get_dispatch():
    return pl.kernel(
        _dispatch_body,
        out_type=jax.ShapeDtypeStruct((NPAD, D2), jnp.int32),
        mesh=_get_sc_mesh(),
        scratch_types=[
            pltpu.VMEM((TOK_W, D2), jnp.int32),
            pltpu.VMEM((TOK_W,), jnp.int32),
            pltpu.VMEM((TOK_W,), jnp.int32),
            pltpu.SemaphoreType.DMA,
        ],
    )


def _wcopies(w1_hbm, w3_hbm, b3_hbm, w2_hbm, w1b, w3b, b3b, w2b, sems,
             e, slot):
    return (
        pltpu.make_async_copy(w1_hbm.at[e], w1b.at[slot], sems.at[slot, 0]),
        pltpu.make_async_copy(w3_hbm.at[e], w3b.at[slot], sems.at[slot, 1]),
        pltpu.make_async_copy(b3_hbm.at[e], b3b.at[slot], sems.at[slot, 2]),
        pltpu.make_async_copy(w2_hbm.at[e], w2b.at[slot], sems.at[slot, 3]),
    )


def _ffn_body(meta_ref, xs_ref, w1_hbm, w3_hbm, b3_hbm, w2_hbm, eo_ref,
              w1b, w3b, b3b, w2b, sems):
    b = pl.program_id(0)
    act = meta_ref[b, 0]
    par = meta_ref[b, 1]
    first = meta_ref[b, 2]
    issue = meta_ref[b, 3]
    nxte = meta_ref[b, 4]
    e_cur = meta_ref[b, 5]

    @pl.when(b == 0)
    def _():
        for cp in _wcopies(w1_hbm, w3_hbm, b3_hbm, w2_hbm,
                           w1b, w3b, b3b, w2b, sems, e_cur, 0):
            cp.start()

    @pl.when(issue == 1)
    def _():
        for cp in _wcopies(w1_hbm, w3_hbm, b3_hbm, w2_hbm,
                           w1b, w3b, b3b, w2b, sems, nxte, 1 - par):
            cp.start()

    @pl.when(first == 1)
    def _():
        for cp in _wcopies(w1_hbm, w3_hbm, b3_hbm, w2_hbm,
                           w1b, w3b, b3b, w2b, sems, e_cur, par):
            cp.wait()

    @pl.when(act == 1)
    def _():
        v = xs_ref[...]
        xlo = lax.bitcast_convert_type(v << 16, jnp.float32).astype(
            jnp.bfloat16)
        xhi = lax.bitcast_convert_type(
            v & jnp.int32(-65536), jnp.float32).astype(jnp.bfloat16)
        w1c = w1b[par].astype(jnp.bfloat16)
        w3c = w3b[par].astype(jnp.bfloat16)
        w2c = w2b[par].astype(jnp.bfloat16)
        cdim = (((1,), (1,)), ((), ()))
        h1 = (lax.dot_general(xlo, w1c[:, :D2], cdim,
                              preferred_element_type=jnp.float32)
              + lax.dot_general(xhi, w1c[:, D2:], cdim,
                                preferred_element_type=jnp.float32))
        h3 = (lax.dot_general(xlo, w3c[:, :D2], cdim,
                              preferred_element_type=jnp.float32)
              + lax.dot_general(xhi, w3c[:, D2:], cdim,
                                preferred_element_type=jnp.float32))
        h3 = h3 + b3b[par]
        h = (h1 * lax.logistic(h1) * h3).astype(jnp.bfloat16)
        eo_ref[...] = lax.dot_general(h, w2c, cdim,
                                      preferred_element_type=jnp.float32)


_ffn_call = pl.pallas_call(
    _ffn_body,
    grid_spec=pltpu.PrefetchScalarGridSpec(
        num_scalar_prefetch=1,
        grid=(NB,),
        in_specs=[
            pl.BlockSpec((T, D2), lambda b, meta: (b, 0)),
            pl.BlockSpec(memory_space=pl.ANY),
            pl.BlockSpec(memory_space=pl.ANY),
            pl.BlockSpec(memory_space=pl.ANY),
            pl.BlockSpec(memory_space=pl.ANY),
        ],
        out_specs=pl.BlockSpec((T, D), lambda b, meta: (b, 0)),
        scratch_shapes=[
            pltpu.VMEM((2, F, D), jnp.float32),
            pltpu.VMEM((2, F, D), jnp.float32),
            pltpu.VMEM((2, 1, F), jnp.float32),
            pltpu.VMEM((2, D, F), jnp.float32),
            pltpu.SemaphoreType.DMA((2, 4)),
        ],
    ),
    out_shape=jax.ShapeDtypeStruct((NPAD, D), jnp.float32),
)


def _combine_body(eo_hbm, slot0_hbm, slot1_hbm, w0_hbm, w1_hbm, out_hbm,
                  idx0_v, idx1_v, w0_v, w1_v, r0_v, r1_v, sem):
    wid = lax.axis_index("s") * 2 + lax.axis_index("c")
    base = wid * TOK_W
    ch = wid // (C // TOK_W)
    off = (wid % (C // TOK_W)) * TOK_W
    pltpu.sync_copy(slot0_hbm.at[ch, pl.ds(off, TOK_W)], idx0_v)
    pltpu.sync_copy(slot1_hbm.at[ch, pl.ds(off, TOK_W)], idx1_v)
    pltpu.sync_copy(w0_hbm.at[pl.ds(base, TOK_W)], w0_v)
    pltpu.sync_copy(w1_hbm.at[pl.ds(base, TOK_W)], w1_v)
    H = TOK_W // 2
    ha, hb = pl.ds(0, H), pl.ds(H, H)
    cA0 = pltpu.async_copy(eo_hbm.at[idx0_v.at[ha]], r0_v.at[ha], sem.at[0])
    cA1 = pltpu.async_copy(eo_hbm.at[idx1_v.at[ha]], r1_v.at[ha], sem.at[0])
    cB0 = pltpu.async_copy(eo_hbm.at[idx0_v.at[hb]], r0_v.at[hb], sem.at[1])
    cB1 = pltpu.async_copy(eo_hbm.at[idx1_v.at[hb]], r1_v.at[hb], sem.at[1])

    def body(i, carry):
        wv0 = w0_v[i]
        wv1 = w1_v[i]
        for j in range(D // LANES):
            sl = pl.ds(j * LANES, LANES)
            r0_v[i, sl] = wv0 * r0_v[i, sl] + wv1 * r1_v[i, sl]
        return carry

    cA0.wait()
    cA1.wait()
    lax.fori_loop(0, H, body, 0)
    oA = pltpu.async_copy(r0_v.at[ha], out_hbm.at[pl.ds(base, H)], sem.at[0])
    cB0.wait()
    cB1.wait()
    lax.fori_loop(H, TOK_W, body, 0)
    oA.wait()
    pltpu.sync_copy(r0_v.at[hb], out_hbm.at[pl.ds(base + H, H)])


@functools.lru_cache(maxsize=None)
def _get_combine():
    return pl.kernel(
        _combine_body,
        out_type=jax.ShapeDtypeStruct((S, D), jnp.float32),
        mesh=_get_sc_mesh(),
        compiler_params=pltpu.CompilerParams(needs_layout_passes=False),
        scratch_types=[
            pltpu.VMEM((TOK_W,), jnp.int32),
            pltpu.VMEM((TOK_W,), jnp.int32),
            pltpu.VMEM((TOK_W, LANES), jnp.float32),
            pltpu.VMEM((TOK_W, LANES), jnp.float32),
            pltpu.VMEM((TOK_W, D), jnp.float32),
            pltpu.VMEM((TOK_W, D), jnp.float32),
            pltpu.SemaphoreType.DMA((2,)),
        ],
    )


@jax.jit
def kernel(x, Wg, W1, W3, b3, W2):
    x2 = x.reshape(S, D)
    slot0, slot1, w0b, w1b, meta, xbf = _router_call(x2, Wg)
    xs = _get_dispatch()(xbf, slot0, slot1)
    eo = _ffn_call(meta, xs, W1, W3, b3.reshape(E, 1, F), W2)
    out = _get_combine()(eo, slot0, slot1, w0b, w1b)
    return out.reshape(1, S, D)


# speedup vs baseline: 1.0394x; 1.0006x over previous
import functools

import jax
import jax.numpy as jnp
from jax import lax
from jax.experimental import pallas as pl
from jax.experimental.pallas import tpu as pltpu
from jax.experimental.pallas import tpu_sc as plsc

S, D, F, E, K = 2048, 768, 1024, 8, 2
T = 256
NB = (S * K) // T + (E - 1)
NPAD = NB * T
MB = 32
C = 256
NCHUNK = S // C
NW = 32
TOK_W = S // NW
LANES = 16
D2 = D // 2


def _router_body(x_ref, wg_ref, slot0_ref, slot1_ref, w0_ref, w1_ref,
                 meta_ref, xbf_ref):
    x = x_ref[...]
    wg = wg_ref[...]
    logits = lax.dot_general(x, wg, (((1,), (1,)), ((), ())),
                             preferred_element_type=jnp.float32)
    eio = lax.broadcasted_iota(jnp.int32, (S, E), 1)
    m0 = jnp.max(logits, axis=1, keepdims=True)
    i0 = jnp.min(jnp.where(logits == m0, eio, E), axis=1, keepdims=True)
    l2 = jnp.where(eio == i0, -jnp.inf, logits)
    m1 = jnp.max(l2, axis=1, keepdims=True)
    i1 = jnp.min(jnp.where(l2 == m1, eio, E), axis=1, keepdims=True)
    w0 = 1.0 / (1.0 + jnp.exp(m1 - m0))
    w1 = 1.0 - w0
    w0_ref[...] = jnp.broadcast_to(w0, (S, LANES))
    w1_ref[...] = jnp.broadcast_to(w1, (S, LANES))
    bits = lax.bitcast_convert_type(x, jnp.uint32)
    blo, bhi = bits[:, :D2], bits[:, D2:]
    rlo = (blo + 0x7FFF + ((blo >> 16) & 1)) >> 16
    rhi = ((bhi + 0x7FFF + ((bhi >> 16) & 1)) >> 16) << 16
    xbf_ref[...] = lax.bitcast_convert_type(rhi | rlo, jnp.int32)

    oh0 = (eio == i0).astype(jnp.float32)
    oh1 = (eio == i1).astype(jnp.float32)
    cnt0 = jnp.sum(oh0, axis=0, keepdims=True)
    cnt = cnt0 + jnp.sum(oh1, axis=0, keepdims=True)
    nblk = jnp.floor((cnt + (T - 1)) * (1.0 / T))
    upper = (lax.broadcasted_iota(jnp.int32, (E, E), 0)
             <= lax.broadcasted_iota(jnp.int32, (E, E), 1)).astype(jnp.float32)
    inc = lax.dot_general(nblk, upper, (((1,), (0,)), ((), ())),
                          preferred_element_type=jnp.float32)
    gs = (inc - nblk) * T

    bio = lax.broadcasted_iota(jnp.int32, (MB, E), 0)
    inc_i = inc.astype(jnp.int32)
    be = jnp.sum((bio >= inc_i).astype(jnp.int32),
                 axis=1, keepdims=True)
    be_p = jnp.sum(((bio - 1) >= inc_i).astype(jnp.int32),
                   axis=1, keepdims=True)
    be = jnp.minimum(be, E - 1)
    be_p = jnp.minimum(be_p, E - 1)
    used = inc[:, E - 1:E]
    bcol = lax.broadcasted_iota(jnp.int32, (MB, 1), 0)
    act = (bcol.astype(jnp.float32) < used).astype(jnp.int32)
    sw = jnp.where((bcol > 0) & (be != be_p), 1, 0) * act
    mtri = (lax.broadcasted_iota(jnp.int32, (MB, MB), 0)
            >= lax.broadcasted_iota(jnp.int32, (MB, MB), 1)).astype(
                jnp.float32)
    run_id = lax.dot_general(mtri, sw.astype(jnp.float32),
                             (((1,), (0,)), ((), ())),
                             preferred_element_type=jnp.float32)
    par = (run_id - 2.0 * jnp.floor(run_id * 0.5)).astype(jnp.int32)
    first = jnp.where((bcol == 0) | (sw == 1), 1, 0) * act
    has = jnp.broadcast_to((nblk > 0.5), (MB, E))
    nxte = jnp.min(jnp.where((eio[:MB] > be) & has, eio[:MB], E),
                   axis=1, keepdims=True)
    issue = first * jnp.where(nxte < E, 1, 0)
    nxte = jnp.minimum(nxte, E - 1)
    meta = jnp.concatenate(
        [act, par, first, issue, nxte, be, be, be], axis=1)
    meta_ref[...] = meta

    ltri = (lax.broadcasted_iota(jnp.int32, (C, C), 0)
            > lax.broadcasted_iota(jnp.int32, (C, C), 1)).astype(jnp.float32)
    carry0 = jnp.zeros((1, E), jnp.float32)
    carry1 = cnt0
    cols0, cols1 = [], []
    for c in range(NCHUNK):
        sl = slice(c * C, (c + 1) * C)
        o0 = oh0[sl]
        o1 = oh1[sl]
        p0 = carry0 + lax.dot_general(ltri, o0, (((1,), (0,)), ((), ())),
                                      preferred_element_type=jnp.float32)
        p1 = carry1 + lax.dot_general(ltri, o1, (((1,), (0,)), ((), ())),
                                      preferred_element_type=jnp.float32)
        carry0 = carry0 + jnp.sum(o0, axis=0, keepdims=True)
        carry1 = carry1 + jnp.sum(o1, axis=0, keepdims=True)
        cols0.append(jnp.sum((p0 + gs) * o0, axis=1, keepdims=True))
        cols1.append(jnp.sum((p1 + gs) * o1, axis=1, keepdims=True))
    slot0_ref[...] = jnp.transpose(
        jnp.concatenate(cols0, axis=1), (1, 0)).astype(jnp.int32)
    slot1_ref[...] = jnp.transpose(
        jnp.concatenate(cols1, axis=1), (1, 0)).astype(jnp.int32)


_router_call = pl.pallas_call(
    _router_body,
    out_shape=(
        jax.ShapeDtypeStruct((NCHUNK, C), jnp.int32),
        jax.ShapeDtypeStruct((NCHUNK, C), jnp.int32),
        jax.ShapeDtypeStruct((S, LANES), jnp.float32),
        jax.ShapeDtypeStruct((S, LANES), jnp.float32),
        jax.ShapeDtypeStruct((MB, 8), jnp.int32),
        jax.ShapeDtypeStruct((S, D2), jnp.int32),
    ),
)


@functools.lru_cache(maxsize=None)
def _get_sc_mesh():
    return plsc.VectorSubcoreMesh(core_axis_name="c", subcore_axis_name="s")


def _dispatch_body(x_hbm, slot0_hbm, slot1_hbm, xs_hbm, rows_v, idx0_v,
                   idx1_v, sem):
    wid = lax.axis_index("s") * 2 + lax.axis_index("c")
    base = wid * TOK_W
    ch = wid // (C // TOK_W)
    off = (wid % (C // TOK_W)) * TOK_W
    pltpu.sync_copy(x_hbm.at[pl.ds(base, TOK_W)], rows_v)
    pltpu.sync_copy(slot0_hbm.at[ch, pl.ds(off, TOK_W)], idx0_v)
    pltpu.sync_copy(slot1_hbm.at[ch, pl.ds(off, TOK_W)], idx1_v)
    pltpu.async_copy(rows_v, xs_hbm.at[idx0_v], sem).wait()
    pltpu.async_copy(rows_v, xs_hbm.at[idx1_v], sem).wait()


@functools.lru_cache(maxsize=None)
def _get_dispatch():
    return pl.kernel(
        _dispatch_body,
        out_type=jax.ShapeDtypeStruct((NPAD, D2), jnp.int32),
        mesh=_get_sc_mesh(),
        scratch_types=[
            pltpu.VMEM((TOK_W, D2), jnp.int32),
            pltpu.VMEM((TOK_W,), jnp.int32),
            pltpu.VMEM((TOK_W,), jnp.int32),
            pltpu.SemaphoreType.DMA,
        ],
    )


def _wcopies(w1_hbm, w3_hbm, b3_hbm, w2_hbm, w1b, w3b, b3b, w2b, sems,
             e, slot):
    return (
        pltpu.make_async_copy(w1_hbm.at[e], w1b.at[slot], sems.at[slot, 0]),
        pltpu.make_async_copy(w3_hbm.at[e], w3b.at[slot], sems.at[slot, 1]),
        pltpu.make_async_copy(b3_hbm.at[e], b3b.at[slot], sems.at[slot, 2]),
        pltpu.make_async_copy(w2_hbm.at[e], w2b.at[slot], sems.at[slot, 3]),
    )


def _ffn_body(meta_ref, xs_ref, w1_hbm, w3_hbm, b3_hbm, w2_hbm, eo_ref,
              w1b, w3b, b3b, w2b, sems):
    b = pl.program_id(0)
    act = meta_ref[b, 0]
    par = meta_ref[b, 1]
    first = meta_ref[b, 2]
    issue = meta_ref[b, 3]
    nxte = meta_ref[b, 4]
    e_cur = meta_ref[b, 5]

    @pl.when(b == 0)
    def _():
        for cp in _wcopies(w1_hbm, w3_hbm, b3_hbm, w2_hbm,
                           w1b, w3b, b3b, w2b, sems, e_cur, 0):
            cp.start()

    @pl.when(issue == 1)
    def _():
        for cp in _wcopies(w1_hbm, w3_hbm, b3_hbm, w2_hbm,
                           w1b, w3b, b3b, w2b, sems, nxte, 1 - par):
            cp.start()

    @pl.when(first == 1)
    def _():
        for cp in _wcopies(w1_hbm, w3_hbm, b3_hbm, w2_hbm,
                           w1b, w3b, b3b, w2b, sems, e_cur, par):
            cp.wait()

    @pl.when(act == 1)
    def _():
        v = xs_ref[...]
        xlo = lax.bitcast_convert_type(v << 16, jnp.float32).astype(
            jnp.bfloat16)
        xhi = lax.bitcast_convert_type(
            v & jnp.int32(-65536), jnp.float32).astype(jnp.bfloat16)
        w1c = w1b[par].astype(jnp.bfloat16)
        w3c = w3b[par].astype(jnp.bfloat16)
        w2c = w2b[par].astype(jnp.bfloat16)
        cdim = (((1,), (1,)), ((), ()))
        h1 = (lax.dot_general(xlo, w1c[:, :D2], cdim,
                              preferred_element_type=jnp.float32)
              + lax.dot_general(xhi, w1c[:, D2:], cdim,
                                preferred_element_type=jnp.float32))
        h3 = (lax.dot_general(xlo, w3c[:, :D2], cdim,
                              preferred_element_type=jnp.float32)
              + lax.dot_general(xhi, w3c[:, D2:], cdim,
                                preferred_element_type=jnp.float32))
        h3 = h3 + b3b[par][None, :]
        h = (h1 * lax.logistic(h1) * h3).astype(jnp.bfloat16)
        eo_ref[...] = lax.dot_general(h, w2c, cdim,
                                      preferred_element_type=jnp.float32)


_ffn_call = pl.pallas_call(
    _ffn_body,
    grid_spec=pltpu.PrefetchScalarGridSpec(
        num_scalar_prefetch=1,
        grid=(NB,),
        in_specs=[
            pl.BlockSpec((T, D2), lambda b, meta: (b, 0)),
            pl.BlockSpec(memory_space=pl.ANY),
            pl.BlockSpec(memory_space=pl.ANY),
            pl.BlockSpec(memory_space=pl.ANY),
            pl.BlockSpec(memory_space=pl.ANY),
        ],
        out_specs=pl.BlockSpec((T, D), lambda b, meta: (b, 0)),
        scratch_shapes=[
            pltpu.VMEM((2, F, D), jnp.float32),
            pltpu.VMEM((2, F, D), jnp.float32),
            pltpu.VMEM((2, F), jnp.float32),
            pltpu.VMEM((2, D, F), jnp.float32),
            pltpu.SemaphoreType.DMA((2, 4)),
        ],
    ),
    out_shape=jax.ShapeDtypeStruct((NPAD, D), jnp.float32),
)


def _combine_body(eo_hbm, slot0_hbm, slot1_hbm, w0_hbm, w1_hbm, out_hbm,
                  idx0_v, idx1_v, w0_v, w1_v, r0_v, r1_v, sem):
    wid = lax.axis_index("s") * 2 + lax.axis_index("c")
    base = wid * TOK_W
    ch = wid // (C // TOK_W)
    off = (wid % (C // TOK_W)) * TOK_W
    pltpu.sync_copy(slot0_hbm.at[ch, pl.ds(off, TOK_W)], idx0_v)
    pltpu.sync_copy(slot1_hbm.at[ch, pl.ds(off, TOK_W)], idx1_v)
    pltpu.sync_copy(w0_hbm.at[pl.ds(base, TOK_W)], w0_v)
    pltpu.sync_copy(w1_hbm.at[pl.ds(base, TOK_W)], w1_v)
    H = TOK_W // 2
    ha, hb = pl.ds(0, H), pl.ds(H, H)
    cA0 = pltpu.async_copy(eo_hbm.at[idx0_v.at[ha]], r0_v.at[ha], sem.at[0])
    cA1 = pltpu.async_copy(eo_hbm.at[idx1_v.at[ha]], r1_v.at[ha], sem.at[0])
    cB0 = pltpu.async_copy(eo_hbm.at[idx0_v.at[hb]], r0_v.at[hb], sem.at[1])
    cB1 = pltpu.async_copy(eo_hbm.at[idx1_v.at[hb]], r1_v.at[hb], sem.at[1])

    def body(i, carry):
        wv0 = w0_v[i]
        wv1 = w1_v[i]
        for j in range(D // LANES):
            sl = pl.ds(j * LANES, LANES)
            r0_v[i, sl] = wv0 * r0_v[i, sl] + wv1 * r1_v[i, sl]
        return carry

    cA0.wait()
    cA1.wait()
    lax.fori_loop(0, H, body, 0)
    oA = pltpu.async_copy(r0_v.at[ha], out_hbm.at[pl.ds(base, H)], sem.at[0])
    cB0.wait()
    cB1.wait()
    lax.fori_loop(H, TOK_W, body, 0)
    oA.wait()
    pltpu.sync_copy(r0_v.at[hb], out_hbm.at[pl.ds(base + H, H)])


@functools.lru_cache(maxsize=None)
def _get_combine():
    return pl.kernel(
        _combine_body,
        out_type=jax.ShapeDtypeStruct((S, D), jnp.float32),
        mesh=_get_sc_mesh(),
        compiler_params=pltpu.CompilerParams(needs_layout_passes=False),
        scratch_types=[
            pltpu.VMEM((TOK_W,), jnp.int32),
            pltpu.VMEM((TOK_W,), jnp.int32),
            pltpu.VMEM((TOK_W, LANES), jnp.float32),
            pltpu.VMEM((TOK_W, LANES), jnp.float32),
            pltpu.VMEM((TOK_W, D), jnp.float32),
            pltpu.VMEM((TOK_W, D), jnp.float32),
            pltpu.SemaphoreType.DMA((2,)),
        ],
    )


@jax.jit
def kernel(x, Wg, W1, W3, b3, W2):
    x2 = x.reshape(S, D)
    slot0, slot1, w0b, w1b, meta, xbf = _router_call(x2, Wg)
    xs = _get_dispatch()(xbf, slot0, slot1)
    eo = _ffn_call(meta, xs, W1, W3, b3, W2)
    out = _get_combine()(eo, slot0, slot1, w0b, w1b)
    return out.reshape(1, S, D)
